# Initial kernel scaffold; baseline (speedup 1.0000x reference)
import jax
import jax.numpy as jnp
from jax.experimental import pallas as pl


def kernel(rpn_class, rpn_bbox, anchors, image):
    raise NotImplementedError("write your pallas kernel here")


# speedup vs baseline: 21.4454x; 21.4454x over previous
import functools

import jax
import jax.numpy as jnp
from jax import lax
from jax.experimental import pallas as pl
from jax.experimental.pallas import tpu as pltpu

_NEG = -1e30
_LANES = 128


def _nms_body(score_ref, dx_ref, dy_ref, dw_ref, dh_ref,
              ax1_ref, ay1_ref, ax2_ref, ay2_ref,
              out_ref,
              x1_ref, y1_ref, x2_ref, y2_ref, area_ref,
              *, k, thr, hmax, wmax):
    s = score_ref[...]

    ax1 = ax1_ref[...]
    ay1 = ay1_ref[...]
    widths = ax2_ref[...] - ax1
    heights = ay2_ref[...] - ay1
    cx = ax1 + 0.5 * widths
    cy = ay1 + 0.5 * heights
    d0 = s * dx_ref[...]
    d1 = s * dy_ref[...]
    d2 = s * dw_ref[...]
    d3 = s * dh_ref[...]
    cx = cx + d0 * widths
    cy = cy + d1 * heights
    nw = widths * jnp.exp(d2)
    nh = heights * jnp.exp(d3)
    x1 = jnp.maximum(cx - 0.5 * nw, 0.0)
    y1 = jnp.maximum(cy - 0.5 * nh, 0.0)
    x2 = jnp.minimum(cx + 0.5 * nw, wmax)
    y2 = jnp.minimum(cy + 0.5 * nh, hmax)
    x1_ref[...] = x1
    y1_ref[...] = y1
    x2_ref[...] = x2
    y2_ref[...] = y2
    area_ref[...] = jnp.maximum(x2 - x1, 0.0) * jnp.maximum(y2 - y1, 0.0)

    idxs = (lax.broadcasted_iota(jnp.int32, s.shape, 0) * _LANES
            + lax.broadcasted_iota(jnp.int32, s.shape, 1))
    lane = lax.broadcasted_iota(jnp.int32, (1, _LANES), 1)

    def body(t, scores):
        m = jnp.max(scores)
        sel = jnp.min(jnp.where(scores == m, idxs, jnp.int32(2 ** 30)))
        selm = idxs == sel
        x1v = x1_ref[...]
        y1v = y1_ref[...]
        x2v = x2_ref[...]
        y2v = y2_ref[...]
        bx1 = jnp.sum(jnp.where(selm, x1v, 0.0))
        by1 = jnp.sum(jnp.where(selm, y1v, 0.0))
        bx2 = jnp.sum(jnp.where(selm, x2v, 0.0))
        by2 = jnp.sum(jnp.where(selm, y2v, 0.0))
        barea = jnp.maximum(bx2 - bx1, 0.0) * jnp.maximum(by2 - by1, 0.0)
        xx1 = jnp.maximum(bx1, x1v)
        yy1 = jnp.maximum(by1, y1v)
        xx2 = jnp.minimum(bx2, x2v)
        yy2 = jnp.minimum(by2, y2v)
        inter = jnp.maximum(xx2 - xx1, 0.0) * jnp.maximum(yy2 - yy1, 0.0)
        iou = inter / (barea + area_ref[...] - inter + 1e-9)
        scores = jnp.where(iou > thr, _NEG, scores)
        scores = jnp.where(selm, _NEG, scores)
        valid = m > _NEG / 2
        row = (jnp.where(lane == 0, bx1, 0.0)
               + jnp.where(lane == 1, by1, 0.0)
               + jnp.where(lane == 2, bx2, 0.0)
               + jnp.where(lane == 3, by2, 0.0))
        out_ref[pl.ds(t, 1), :] = jnp.where(valid, row, 0.0)
        return scores

    lax.fori_loop(0, k, body, s)


def _pad2d(flat, rows, fill):
    n = flat.shape[0]
    pad = rows * _LANES - n
    return jnp.pad(flat, (0, pad), constant_values=fill).reshape(rows, _LANES)


def kernel(rpn_class, rpn_bbox, anchors, image):
    n = rpn_bbox.shape[0]
    k = 1000
    thr = 0.7
    rows = (n + _LANES - 1) // _LANES
    rows = ((rows + 7) // 8) * 8

    scores = _pad2d(jnp.reshape(rpn_class, (-1,)), rows, _NEG)
    dx = _pad2d(rpn_bbox[:, 0], rows, 0.0)
    dy = _pad2d(rpn_bbox[:, 1], rows, 0.0)
    dw = _pad2d(rpn_bbox[:, 2], rows, 0.0)
    dh = _pad2d(rpn_bbox[:, 3], rows, 0.0)
    ax1 = _pad2d(anchors[:, 0], rows, 0.0)
    ay1 = _pad2d(anchors[:, 1], rows, 0.0)
    ax2 = _pad2d(anchors[:, 2], rows, 0.0)
    ay2 = _pad2d(anchors[:, 3], rows, 0.0)

    body = functools.partial(
        _nms_body, k=k, thr=thr,
        hmax=float(image.shape[0] - 1), wmax=float(image.shape[1] - 1))
    out = pl.pallas_call(
        body,
        out_shape=jax.ShapeDtypeStruct((k, _LANES), jnp.float32),
        scratch_shapes=[pltpu.VMEM((rows, _LANES), jnp.float32)] * 5,
    )(scores, dx, dy, dw, dh, ax1, ay1, ax2, ay2)
    return out[:, :4]


# speedup vs baseline: 25.3365x; 1.1814x over previous
import functools

import jax
import jax.numpy as jnp
from jax import lax
from jax.experimental import pallas as pl
from jax.experimental.pallas import tpu as pltpu

_NEG = -1e30
_LANES = 128
_KSW = 4


def _nms_body(score_ref, dx_ref, dy_ref, dw_ref, dh_ref,
              ax1_ref, ay1_ref, ax2_ref, ay2_ref,
              out_ref,
              x1_ref, y1_ref, x2_ref, y2_ref, area_ref,
              *, k, thr, hmax, wmax):
    s = score_ref[...]

    ax1 = ax1_ref[...]
    ay1 = ay1_ref[...]
    widths = ax2_ref[...] - ax1
    heights = ay2_ref[...] - ay1
    cx = ax1 + 0.5 * widths
    cy = ay1 + 0.5 * heights
    d0 = s * dx_ref[...]
    d1 = s * dy_ref[...]
    d2 = s * dw_ref[...]
    d3 = s * dh_ref[...]
    cx = cx + d0 * widths
    cy = cy + d1 * heights
    nw = widths * jnp.exp(d2)
    nh = heights * jnp.exp(d3)
    x1 = jnp.maximum(cx - 0.5 * nw, 0.0)
    y1 = jnp.maximum(cy - 0.5 * nh, 0.0)
    x2 = jnp.minimum(cx + 0.5 * nw, wmax)
    y2 = jnp.minimum(cy + 0.5 * nh, hmax)
    x1_ref[...] = x1
    y1_ref[...] = y1
    x2_ref[...] = x2
    y2_ref[...] = y2
    area_ref[...] = jnp.maximum(x2 - x1, 0.0) * jnp.maximum(y2 - y1, 0.0)

    out_ref[...] = jnp.zeros((k, _LANES), jnp.float32)

    idxs = (lax.broadcasted_iota(jnp.int32, s.shape, 0) * _LANES
            + lax.broadcasted_iota(jnp.int32, s.shape, 1))
    lane = lax.broadcasted_iota(jnp.int32, (1, _LANES), 1)

    def sweep(state):
        scores, cnt, _ = state
        tmp = scores
        ms, sels, selms = [], [], []
        for _j in range(_KSW):
            mj = jnp.max(tmp)
            selj = jnp.min(jnp.where(tmp == mj, idxs, jnp.int32(2 ** 30)))
            selmj = idxs == selj
            tmp = jnp.where(selmj, _NEG, tmp)
            ms.append(mj)
            sels.append(selj)
            selms.append(selmj)

        x1v = x1_ref[...]
        y1v = y1_ref[...]
        x2v = x2_ref[...]
        y2v = y2_ref[...]
        areav = area_ref[...]
        cols, rows_out = [], []
        for j in range(_KSW):
            r = sels[j] // _LANES
            lanem = lane == (sels[j] % _LANES)
            bx1 = jnp.sum(jnp.where(lanem, x1_ref[pl.ds(r, 1), :], 0.0))
            by1 = jnp.sum(jnp.where(lanem, y1_ref[pl.ds(r, 1), :], 0.0))
            bx2 = jnp.sum(jnp.where(lanem, x2_ref[pl.ds(r, 1), :], 0.0))
            by2 = jnp.sum(jnp.where(lanem, y2_ref[pl.ds(r, 1), :], 0.0))
            barea = (jnp.maximum(bx2 - bx1, 0.0) *
                     jnp.maximum(by2 - by1, 0.0))
            xx1 = jnp.maximum(bx1, x1v)
            yy1 = jnp.maximum(by1, y1v)
            xx2 = jnp.minimum(bx2, x2v)
            yy2 = jnp.minimum(by2, y2v)
            inter = (jnp.maximum(xx2 - xx1, 0.0) *
                     jnp.maximum(yy2 - yy1, 0.0))
            cols.append(inter / (barea + areav - inter + 1e-9) > thr)
            rows_out.append(jnp.where(lane == 0, bx1, 0.0)
                            + jnp.where(lane == 1, by1, 0.0)
                            + jnp.where(lane == 2, bx2, 0.0)
                            + jnp.where(lane == 3, by2, 0.0))

        sup = {}
        for i in range(_KSW):
            for j in range(i + 1, _KSW):
                sup[(i, j)] = jnp.max(
                    jnp.where(selms[j] & cols[i], 1.0, 0.0)) > 0.0

        acc = []
        for j in range(_KSW):
            aj = ms[j] > _NEG / 2
            for i in range(j):
                aj = aj & ~(acc[i] & sup[(i, j)])
            acc.append(aj)

        kill = jnp.zeros_like(scores, dtype=jnp.bool_)
        for j in range(_KSW):
            kill = kill | (acc[j] & (cols[j] | selms[j]))
        scores = jnp.where(kill, _NEG, scores)

        for j in range(_KSW):
            do = acc[j] & (cnt < k)

            @pl.when(do)
            def _store(j=j, cnt=cnt):
                out_ref[pl.ds(cnt, 1), :] = rows_out[j]

            cnt = cnt + jnp.where(do, 1, 0)

        go = (cnt < k) & acc[0]
        return scores, cnt, go

    lax.while_loop(lambda st: st[2], sweep,
                   (s, jnp.int32(0), jnp.bool_(True)))


def _pad2d(flat, rows, fill):
    n = flat.shape[0]
    pad = rows * _LANES - n
    return jnp.pad(flat, (0, pad), constant_values=fill).reshape(rows, _LANES)


def kernel(rpn_class, rpn_bbox, anchors, image):
    n = rpn_bbox.shape[0]
    k = 1000
    thr = 0.7
    rows = (n + _LANES - 1) // _LANES
    rows = ((rows + 7) // 8) * 8

    scores = _pad2d(jnp.reshape(rpn_class, (-1,)), rows, _NEG)
    dx = _pad2d(rpn_bbox[:, 0], rows, 0.0)
    dy = _pad2d(rpn_bbox[:, 1], rows, 0.0)
    dw = _pad2d(rpn_bbox[:, 2], rows, 0.0)
    dh = _pad2d(rpn_bbox[:, 3], rows, 0.0)
    ax1 = _pad2d(anchors[:, 0], rows, 0.0)
    ay1 = _pad2d(anchors[:, 1], rows, 0.0)
    ax2 = _pad2d(anchors[:, 2], rows, 0.0)
    ay2 = _pad2d(anchors[:, 3], rows, 0.0)

    body = functools.partial(
        _nms_body, k=k, thr=thr,
        hmax=float(image.shape[0] - 1), wmax=float(image.shape[1] - 1))
    out = pl.pallas_call(
        body,
        out_shape=jax.ShapeDtypeStruct((k, _LANES), jnp.float32),
        scratch_shapes=[pltpu.VMEM((rows, _LANES), jnp.float32)] * 5,
    )(scores, dx, dy, dw, dh, ax1, ay1, ax2, ay2)
    return out[:, :4]


# speedup vs baseline: 25.3532x; 1.0007x over previous
import functools

import jax
import jax.numpy as jnp
from jax import lax
from jax.experimental import pallas as pl
from jax.experimental.pallas import tpu as pltpu
from jax.experimental.pallas import tpu_sc as plsc

_NEG = -1e30
_L = 16
_NT = 16
_BIG = 2 ** 30


def _shuf(v, idx):
    return v.at[idx].get(mode="promise_in_bounds")


def _csplat(c):
    return jnp.broadcast_to(jnp.int32(c), (_L,))


def _bfly_max(v, lane):
    for s in (1, 2, 4, 8):
        v = jnp.maximum(v, _shuf(v, lane ^ s))
    return v


def _bfly_min(v, lane):
    for s in (1, 2, 4, 8):
        v = jnp.minimum(v, _shuf(v, lane ^ s))
    return v


def _sc_body(score_h, dx_h, dy_h, dw_h, dh_h, ax1_h, ay1_h, ax2_h, ay2_h,
             out_h,
             sco_v, x1_v, y1_v, x2_v, y2_v, ar_v,
             sa_v, sb_v, sc2_v, sd_v,
             publoc_v, pubbuf_v, ibuf_v, outbuf_v, pub_sh,
             *, k, thr, hmax, wmax, per_tile):
    cid = lax.axis_index("c")
    sid = lax.axis_index("s")
    nchunk = per_tile // _L
    lane = lax.iota(jnp.int32, _L)

    @pl.when(cid == 0)
    def _main():
        base = sid * per_tile
        pltpu.sync_copy(score_h.at[pl.ds(base, per_tile)], sco_v)
        pltpu.sync_copy(dx_h.at[pl.ds(base, per_tile)], sa_v)
        pltpu.sync_copy(dy_h.at[pl.ds(base, per_tile)], sb_v)
        pltpu.sync_copy(ax1_h.at[pl.ds(base, per_tile)], sc2_v)
        pltpu.sync_copy(ay1_h.at[pl.ds(base, per_tile)], sd_v)
        pltpu.sync_copy(ax2_h.at[pl.ds(base, per_tile)], x1_v)
        pltpu.sync_copy(ay2_h.at[pl.ds(base, per_tile)], y1_v)
        for i in range(nchunk):
            ds = pl.ds(i * _L, _L)
            s = sco_v[ds]
            a1 = sc2_v[ds]
            b1 = sd_v[ds]
            w = x1_v[ds] - a1
            h = y1_v[ds] - b1
            cx = a1 + 0.5 * w
            cy = b1 + 0.5 * h
            cx = cx + (s * sa_v[ds]) * w
            cy = cy + (s * sb_v[ds]) * h
            x2_v[ds] = cx
            y2_v[ds] = cy
        pltpu.sync_copy(dw_h.at[pl.ds(base, per_tile)], sa_v)
        pltpu.sync_copy(dh_h.at[pl.ds(base, per_tile)], sb_v)
        for i in range(nchunk):
            ds = pl.ds(i * _L, _L)
            s = sco_v[ds]
            w = (x1_v[ds] - sc2_v[ds]) * jnp.exp(s * sa_v[ds])
            h = (y1_v[ds] - sd_v[ds]) * jnp.exp(s * sb_v[ds])
            cx = x2_v[ds]
            cy = y2_v[ds]
            nx1 = jnp.maximum(cx - 0.5 * w, 0.0)
            ny1 = jnp.maximum(cy - 0.5 * h, 0.0)
            nx2 = jnp.minimum(cx + 0.5 * w, wmax)
            ny2 = jnp.minimum(cy + 0.5 * h, hmax)
            x1_v[ds] = nx1
            y1_v[ds] = ny1
            x2_v[ds] = nx2
            y2_v[ds] = ny2
            ar_v[ds] = jnp.maximum(nx2 - nx1, 0.0) * jnp.maximum(ny2 - ny1, 0.0)

        lanef = lane.astype(jnp.float32)
        basef = base.astype(jnp.float32)
        bigf = jnp.float32(2.0 ** 30)

        def step(t, carry):
            m = jnp.broadcast_to(jnp.float32(_NEG), (_L,))
            mif = jnp.broadcast_to(jnp.float32(0.0), (_L,))
            for i in range(nchunk):
                v = sco_v[pl.ds(i * _L, _L)]
                upd = v > m
                m = jnp.where(upd, v, m)
                mif = jnp.where(upd, jnp.float32(i), mif)
            mm = _bfly_max(m, lane)
            gidxf = mif * float(_L) + lanef + basef
            lwinf = _bfly_min(jnp.where(m == mm, gidxf, bigf), lane)
            lofsi = (lwinf - basef).astype(jnp.int32)
            lofs_s = lofsi[0]
            start = pl.multiple_of((lofs_s // _L) * _L, _L)
            lsel = lofsi % _L
            bx1 = _shuf(x1_v[pl.ds(start, _L)], lsel)
            by1 = _shuf(y1_v[pl.ds(start, _L)], lsel)
            bx2 = _shuf(x2_v[pl.ds(start, _L)], lsel)
            by2 = _shuf(y2_v[pl.ds(start, _L)], lsel)
            pv = jnp.where(lanef == 0.0, mm, lwinf)
            pv = jnp.where(lanef == 2.0, bx1, pv)
            pv = jnp.where(lanef == 3.0, by1, pv)
            pv = jnp.where(lanef == 4.0, bx2, pv)
            pv = jnp.where(lanef == 5.0, by2, pv)
            publoc_v[...] = pv
            plsc.subcore_barrier()
            pltpu.sync_copy(publoc_v, pub_sh.at[pl.ds(sid * _L, _L)])
            plsc.subcore_barrier()
            pltpu.sync_copy(pub_sh, pubbuf_v)
            c0 = _csplat(0)
            c1 = _csplat(1)
            gm = jnp.broadcast_to(jnp.float32(_NEG), (_L,))
            for tt in range(_NT):
                gm = jnp.maximum(gm, _shuf(pubbuf_v[pl.ds(tt * _L, _L)], c0))
            gif = bigf * jnp.broadcast_to(jnp.float32(1.0), (_L,))
            for tt in range(_NT):
                r = pubbuf_v[pl.ds(tt * _L, _L)]
                bm = _shuf(r, c0)
                bif = _shuf(r, c1)
                gif = jnp.minimum(gif, jnp.where(bm == gm, bif, bigf))
            c2 = _csplat(2)
            c3 = _csplat(3)
            c4 = _csplat(4)
            c5 = _csplat(5)
            zz = jnp.broadcast_to(jnp.float32(0.0), (_L,))
            gx1 = zz
            gy1 = zz
            gx2 = zz
            gy2 = zz
            for tt in range(_NT):
                r = pubbuf_v[pl.ds(tt * _L, _L)]
                hit = _shuf(r, c1) == gif
                gx1 = jnp.where(hit, _shuf(r, c2), gx1)
                gy1 = jnp.where(hit, _shuf(r, c3), gy1)
                gx2 = jnp.where(hit, _shuf(r, c4), gx2)
                gy2 = jnp.where(hit, _shuf(r, c5), gy2)
            barea = (jnp.maximum(gx2 - gx1, 0.0) *
                     jnp.maximum(gy2 - gy1, 0.0))
            for i in range(nchunk):
                ds = pl.ds(i * _L, _L)
                xx1 = jnp.maximum(gx1, x1_v[ds])
                yy1 = jnp.maximum(gy1, y1_v[ds])
                xx2 = jnp.minimum(gx2, x2_v[ds])
                yy2 = jnp.minimum(gy2, y2_v[ds])
                inter = (jnp.maximum(xx2 - xx1, 0.0) *
                         jnp.maximum(yy2 - yy1, 0.0))
                iou = inter / (barea + ar_v[ds] - inter + 1e-9)
                gidxcf = (jnp.broadcast_to(jnp.float32(base + i * _L), (_L,))
                          + lanef)
                sc = jnp.where(iou > thr, _NEG, sco_v[ds])
                sco_v[ds] = jnp.where(gidxcf == gif, _NEG, sc)

            @pl.when(sid == 0)
            def _out():
                validf = jnp.where(gm > _NEG / 2,
                                   jnp.broadcast_to(jnp.float32(1.0), (_L,)),
                                   jnp.broadcast_to(jnp.float32(0.0), (_L,)))
                row = jnp.where(lanef == 0.0, gx1, 0.0)
                row = jnp.where(lanef == 1.0, gy1, row)
                row = jnp.where(lanef == 2.0, gx2, row)
                row = jnp.where(lanef == 3.0, gy2, row)
                outbuf_v[pl.ds(t * _L, _L)] = row * validf
            return carry

        lax.fori_loop(0, k, step, jnp.int32(0))

        @pl.when(sid == 0)
        def _flush():
            pltpu.sync_copy(outbuf_v, out_h)


def _pad1d(flat, np_, fill):
    return jnp.pad(flat, (0, np_ - flat.shape[0]), constant_values=fill)


def kernel(rpn_class, rpn_bbox, anchors, image):
    n = rpn_bbox.shape[0]
    k = 1000
    thr = 0.7
    per_tile = ((n + _NT * _L - 1) // (_NT * _L)) * _L
    np_ = per_tile * _NT

    scores = _pad1d(jnp.reshape(rpn_class, (-1,)), np_, _NEG)
    cols = [_pad1d(rpn_bbox[:, i], np_, 0.0) for i in range(4)]
    acols = [_pad1d(anchors[:, i], np_, 0.0) for i in range(4)]

    mesh = plsc.VectorSubcoreMesh(core_axis_name="c", subcore_axis_name="s")
    body = functools.partial(
        _sc_body, k=k, thr=thr,
        hmax=float(image.shape[0] - 1), wmax=float(image.shape[1] - 1),
        per_tile=per_tile)
    f = pl.kernel(
        body,
        mesh=mesh,
        out_type=jax.ShapeDtypeStruct((k * _L,), jnp.float32),
        scratch_types=[pltpu.VMEM((per_tile,), jnp.float32)] * 10
        + [pltpu.VMEM((_L,), jnp.float32),
           pltpu.VMEM((_NT * _L,), jnp.float32),
           pltpu.VMEM((_L,), jnp.int32),
           pltpu.VMEM((k * _L,), jnp.float32),
           pltpu.VMEM_SHARED((_NT * _L,), jnp.float32)],
    )
    out = f(scores, *cols, *acols)
    return out.reshape(k, _L)[:, :4]
